# baseline (speedup 1.0000x reference)
import functools

import jax
import jax.numpy as jnp
from jax import lax
from jax.experimental import pallas as pl
from jax.experimental.pallas import tpu as pltpu

Q = 1024
D = 16
N = 100000
NT = 2048
NPAD = 100352
NEG = -3.0e38


def _matmul_body(q_ref, k_ref, o_ref):
    j = pl.program_id(0)
    s = lax.dot_general(
        q_ref[...], k_ref[...], (((1,), (1,)), ((), ())),
        preferred_element_type=jnp.float32)
    col = j * NT + lax.broadcasted_iota(jnp.int32, (Q, NT), 1)
    o_ref[...] = jnp.where(col < N, s, NEG)


def _scores(queries, keys_pad):
    return pl.pallas_call(
        _matmul_body,
        grid=(NPAD // NT,),
        in_specs=[
            pl.BlockSpec((Q, D), lambda j: (0, 0)),
            pl.BlockSpec((NT, D), lambda j: (j, 0)),
        ],
        out_specs=pl.BlockSpec((Q, NT), lambda j: (0, j)),
        out_shape=jax.ShapeDtypeStruct((Q, NPAD), jnp.float32),
    )(queries, keys_pad)


def kernel(queries, keys, k):
    keys_pad = jnp.pad(keys, ((0, NPAD - N), (0, 0)))
    scores = _scores(queries, keys_pad)
    values, indices = lax.top_k(scores[:, :N], 100)
    indices = indices + (jnp.asarray(k, dtype=indices.dtype) - 100)
    return values, indices


# speedup vs baseline: 24.2983x; 24.2983x over previous
import functools

import jax
import jax.numpy as jnp
from jax import lax
from jax.experimental import pallas as pl
from jax.experimental.pallas import tpu as pltpu
from jax.experimental.pallas import tpu_sc as plsc

Q = 1024
D = 16
N = 100000
NT = 2048
NPAD = 100352
K = 100
KPAD = 128
NEG = -3.0e38

L = 16
NLEAF = NPAD // L
S1 = 16
NB1 = NLEAF // S1
S2 = 14
NB2 = NB1 // S2
NWORKERS = 32
QPW = Q // NWORKERS


def _matmul_body(q_ref, k_ref, o_ref):
    j = pl.program_id(0)
    s = lax.dot_general(
        q_ref[...], k_ref[...], (((1,), (1,)), ((), ())),
        preferred_element_type=jnp.float32)
    col = j * NT + lax.broadcasted_iota(jnp.int32, (Q, NT), 1)
    o_ref[...] = jnp.where(col < N, s, NEG)


def _scores(queries, keys_pad):
    return pl.pallas_call(
        _matmul_body,
        grid=(NPAD // NT,),
        in_specs=[
            pl.BlockSpec((Q, D), lambda j: (0, 0)),
            pl.BlockSpec((NT, D), lambda j: (j, 0)),
        ],
        out_specs=pl.BlockSpec((Q, NT), lambda j: (0, j)),
        out_shape=jax.ShapeDtypeStruct((Q, NPAD), jnp.float32),
    )(queries, keys_pad)


def _store1(ref, pos, val, iota):
    blk = (pos // L) * L
    vv = ref[pl.ds(blk, L)]
    ref[pl.ds(blk, L)] = jnp.where(iota == pos - blk, val, vv)


def _topk_sc(scores):
    mesh = plsc.VectorSubcoreMesh(core_axis_name="c", subcore_axis_name="s")

    @functools.partial(
        pl.kernel,
        mesh=mesh,
        out_type=(jax.ShapeDtypeStruct((Q, KPAD), jnp.float32),
                  jax.ShapeDtypeStruct((Q, KPAD), jnp.int32)),
        scratch_types=[
            pltpu.VMEM((NPAD,), jnp.float32),
            pltpu.VMEM((NB1 * L,), jnp.float32),
            pltpu.VMEM((NB1 * L,), jnp.int32),
            pltpu.VMEM((NB2 * L,), jnp.float32),
            pltpu.VMEM((NB2 * L,), jnp.int32),
            pltpu.VMEM((KPAD,), jnp.float32),
            pltpu.VMEM((KPAD,), jnp.int32),
            pltpu.VMEM((L,), jnp.int32),
        ],
    )
    def run(scores_hbm, outv_hbm, outi_hbm, row, l1v, l1e, l2v, l2e, ov, oi,
            sb):
        wid = lax.axis_index("s") * 2 + lax.axis_index("c")
        iota = lax.iota(jnp.int32, L)
        shufs = [jnp.bitwise_xor(iota, s) for s in (8, 4, 2, 1)]

        dnums = lax.GatherDimensionNumbers(
            offset_dims=(), collapsed_slice_dims=(0,), start_index_map=(0,))

        def shuffle(v, idx):
            return lax.gather(
                v, idx[:, None], dnums, (1,),
                mode=lax.GatherScatterMode.PROMISE_IN_BOUNDS)

        def allmax(v):
            for idx in shufs:
                v = jnp.maximum(v, shuffle(v, idx))
            return v

        def allmin(v):
            for idx in shufs:
                v = jnp.minimum(v, shuffle(v, idx))
            return v

        def rebuild_l1(b):
            base = b * (S1 * L)
            mval = row[pl.ds(base, L)]
            menc = jnp.full((L,), b * S1, jnp.int32)
            for i in range(1, S1):
                v = row[pl.ds(base + i * L, L)]
                gt = v > mval
                mval = jnp.where(gt, v, mval)
                menc = jnp.where(gt, b * S1 + i, menc)
            l1v[pl.ds(b * L, L)] = mval
            l1e[pl.ds(b * L, L)] = menc

        def rebuild_l2(c):
            base = c * S2 * L
            cval = l1v[pl.ds(base, L)]
            cenc = l1e[pl.ds(base, L)]
            for t in range(1, S2):
                v = l1v[pl.ds(base + t * L, L)]
                en = l1e[pl.ds(base + t * L, L)]
                gt = v > cval
                cval = jnp.where(gt, v, cval)
                cenc = jnp.where(gt, en, cenc)
            l2v[pl.ds(c * L, L)] = cval
            l2e[pl.ds(c * L, L)] = cenc

        def do_query(qi, _):
            q = wid * QPW + qi
            pltpu.sync_copy(scores_hbm.at[q], row)

            def b_l1(b, _):
                rebuild_l1(b)
                return 0
            lax.fori_loop(0, NB1, b_l1, 0)

            def b_l2(c, _):
                rebuild_l2(c)
                return 0
            lax.fori_loop(0, NB2, b_l2, 0)

            def extract(e, _):
                rval = l2v[pl.ds(0, L)]
                renc = l2e[pl.ds(0, L)]
                for cc in range(1, NB2):
                    v = l2v[pl.ds(cc * L, L)]
                    en = l2e[pl.ds(cc * L, L)]
                    gt = v > rval
                    rval = jnp.where(gt, v, rval)
                    renc = jnp.where(gt, en, renc)
                mvec = allmax(rval)
                nvec = allmin(
                    jnp.where(rval == mvec, renc * L + iota, 0x7FFFFFFF))
                n = nvec[0]
                j = n // L
                lane = n - j * L
                vv = row[pl.ds(j * L, L)]
                row[pl.ds(j * L, L)] = jnp.where(iota == lane, NEG, vv)
                b = j // S1
                rebuild_l1(b)
                rebuild_l2(b // S2)
                _store1(ov, e, mvec, iota)
                _store1(oi, e, nvec, iota)
                return 0
            lax.fori_loop(0, K, extract, 0)

            pltpu.sync_copy(ov, outv_hbm.at[q])
            pltpu.sync_copy(oi, outi_hbm.at[q])
            return 0

        for t in range(KPAD // L):
            ov[pl.ds(t * L, L)] = jnp.zeros((L,), jnp.float32)
            oi[pl.ds(t * L, L)] = jnp.zeros((L,), jnp.int32)
        lax.fori_loop(0, QPW, do_query, 0)

    return run(scores)


def kernel(queries, keys, k):
    keys_pad = jnp.pad(keys, ((0, NPAD - N), (0, 0)))
    scores = _scores(queries, keys_pad)
    vals, idxs = _topk_sc(scores)
    values = vals[:, :K]
    indices = idxs[:, :K] + (jnp.asarray(k, dtype=jnp.int32) - K)
    return values, indices
